# baseline (speedup 1.0000x reference)
import functools

import jax
import jax.numpy as jnp
from jax import lax
from jax.experimental import pallas as pl
from jax.experimental.pallas import tpu as pltpu
from jax.experimental.pallas import tpu_sc as plsc

N_NODES = 10000
CH = 128
N_SP = 10240
RPT = N_SP // 16
EPB = 128
NBT = 80
NW = 32
E_PAD = NW * NBT * EPB

_mesh = plsc.VectorSubcoreMesh(core_axis_name="c", subcore_axis_name="s")


EPT = E_PAD // NW
NR = 80


@functools.partial(
    pl.kernel,
    out_type=jax.ShapeDtypeStruct((2, NR, 128), jnp.float32),
    mesh=_mesh,
    scratch_types=[
        pltpu.VMEM((EPT,), jnp.int32),
        pltpu.VMEM((NR, 128), jnp.float32),
        pltpu.VMEM_SHARED((NR, 128), jnp.float32),
    ],
    compiler_params=pltpu.CompilerParams(needs_layout_passes=False),
)
def _sc_degree(dst_hbm, zeros_hbm, out_hbm, dst_v, hist_v, acc_sh):
    c = lax.axis_index("c")
    s = lax.axis_index("s")
    wid = s * 2 + c
    pltpu.sync_copy(zeros_hbm, hist_v)
    pltpu.sync_copy(dst_hbm.at[pl.ds(wid * EPT, EPT)], dst_v)

    @pl.when(s == 0)
    def _():
        pltpu.sync_copy(zeros_hbm, acc_sh)

    plsc.subcore_barrier()

    def body(i, carry):
        idx = dst_v[pl.ds(i * 16, 16)]
        cnt, last = plsc.scan_count(idx)
        row = lax.shift_right_logical(idx, 7)
        col = lax.bitwise_and(idx, 127)
        plsc.addupdate_scatter(hist_v, [row, col], cnt.astype(jnp.float32),
                               mask=last)
        return carry

    lax.fori_loop(0, EPT // 16, body, 0, unroll=4)
    def combine(k, carry):
        rows = jnp.arange(16, dtype=jnp.int32) + k * 16
        pltpu.sync_copy(hist_v.at[pl.ds(k * 16, 16)], acc_sh.at[rows], add=True)
        return carry

    lax.fori_loop(0, NR // 16, combine, 0)
    plsc.subcore_barrier()

    @pl.when(s < 10)
    def _():
        pltpu.sync_copy(acc_sh.at[pl.ds(s * 8, 8)],
                        out_hbm.at[c, pl.ds(s * 8, 8)])


INNER = 16
DEPTH = 2
BF = 80


@functools.partial(
    pl.kernel,
    out_type=jax.ShapeDtypeStruct((2, N_SP, CH), jnp.float32),
    mesh=_mesh,
    scratch_types=[
        pltpu.VMEM((INNER, EPB), jnp.int32),
        pltpu.VMEM((INNER, EPB), jnp.int32),
        pltpu.VMEM((EPB, CH), jnp.float32),
        pltpu.VMEM((EPB, CH), jnp.float32),
        pltpu.VMEM_SHARED((N_SP, CH), jnp.float32),
        pltpu.SemaphoreType.DMA,
        pltpu.SemaphoreType.DMA,
    ],
)
def _sc_scatter(h_hbm, src_hbm, dst_hbm, zeros_hbm, out_hbm,
                src_v, dst_v, b0, b1, acc_sh, s0, s1):
    bufs = (b0, b1)
    sems = (s0, s1)
    c = lax.axis_index("c")
    s = lax.axis_index("s")
    base = s * RPT
    with jax.named_scope("zero_acc"):
        pltpu.sync_copy(zeros_hbm.at[pl.ds(base, RPT)], acc_sh.at[pl.ds(base, RPT)])
        plsc.subcore_barrier()

    def chunk(row0):
        pltpu.sync_copy(src_hbm.at[pl.ds(row0, INNER)], src_v)
        pltpu.sync_copy(dst_hbm.at[pl.ds(row0, INNER)], dst_v)
        descs = [None] * INNER
        for b in range(DEPTH):
            descs[b] = pltpu.async_copy(h_hbm.at[src_v.at[b]], bufs[b], sems[b])
        for b in range(INNER):
            descs[b].wait()
            pltpu.sync_copy(bufs[b % DEPTH], acc_sh.at[dst_v.at[b]], add=True)
            if b + DEPTH < INNER:
                nb = b + DEPTH
                descs[nb] = pltpu.async_copy(
                    h_hbm.at[src_v.at[nb]], bufs[nb % DEPTH], sems[nb % DEPTH])

    with jax.named_scope("edge_loop"):
        @pl.when(c == 0)
        def _():
            def body0(g, carry):
                chunk(s * 2 * NBT + g * INNER)
                return carry
            lax.fori_loop(0, BF // INNER, body0, 0)

        @pl.when(c == 1)
        def _():
            def body1(g, carry):
                chunk(s * 2 * NBT + BF + g * INNER)
                return carry
            lax.fori_loop(0, (2 * NBT - BF) // INNER, body1, 0)

    with jax.named_scope("drain"):
        plsc.subcore_barrier()
    with jax.named_scope("out_copy"):
        pltpu.sync_copy(acc_sh.at[pl.ds(base, RPT)], out_hbm.at[c, pl.ds(base, RPT)])


NBLK = 8
BRS = N_SP // NBLK
NRB = N_SP // 128


def _tc_a_body(deg_ref, x_ref, w_ref, h_ref, dinv_ref):
    deg = deg_ref[0] + deg_ref[1] + 1.0
    dinv8 = lax.rsqrt(deg)
    h = jnp.dot(x_ref[...], w_ref[...], preferred_element_type=jnp.float32)
    for k in range(8):
        col = dinv8[k:k + 1, :].reshape(128, 1)
        dinv_ref[pl.ds(k * 128, 128), :] = col
        h_ref[pl.ds(k * 128, 128), :] = h[k * 128:(k + 1) * 128, :] * col


_tc_a = pl.pallas_call(
    _tc_a_body,
    grid=(10,),
    in_specs=[
        pl.BlockSpec((2, 8, 128), lambda i: (0, i, 0)),
        pl.BlockSpec((1024, CH), lambda i: (i, 0)),
        pl.BlockSpec((CH, CH), lambda i: (0, 0)),
    ],
    out_specs=(
        pl.BlockSpec((1024, CH), lambda i: (i, 0)),
        pl.BlockSpec((1024, 1), lambda i: (i, 0)),
    ),
    out_shape=(
        jax.ShapeDtypeStruct((N_SP, CH), jnp.float32),
        jax.ShapeDtypeStruct((N_SP, 1), jnp.float32),
    ),
)


def _tc_b_body(acc_ref, h_ref, dinv_ref, b_ref, w_ref, out_ref):
    dinv = dinv_ref[...]
    z = (acc_ref[0] + acc_ref[1] + h_ref[...]) * dinv + b_ref[...]
    z = jnp.maximum(z, 0.0)
    out_ref[...] = jnp.dot(z, w_ref[...], preferred_element_type=jnp.float32) * dinv


_tc_b = pl.pallas_call(
    _tc_b_body,
    grid=(NBLK,),
    in_specs=[
        pl.BlockSpec((2, BRS, CH), lambda i: (0, i, 0)),
        pl.BlockSpec((BRS, CH), lambda i: (i, 0)),
        pl.BlockSpec((BRS, 1), lambda i: (i, 0)),
        pl.BlockSpec((1, CH), lambda i: (0, 0)),
        pl.BlockSpec((CH, CH), lambda i: (0, 0)),
    ],
    out_specs=pl.BlockSpec((BRS, CH), lambda i: (i, 0)),
    out_shape=jax.ShapeDtypeStruct((N_SP, CH), jnp.float32),
)


def _tc_c_body(acc_ref, h_ref, dinv_ref, b_ref, out_ref):
    out_ref[...] = (acc_ref[0] + acc_ref[1] + h_ref[...]) * dinv_ref[...] + b_ref[...]


_tc_c = pl.pallas_call(
    _tc_c_body,
    grid=(NBLK,),
    in_specs=[
        pl.BlockSpec((2, BRS, CH), lambda i: (0, i, 0)),
        pl.BlockSpec((BRS, CH), lambda i: (i, 0)),
        pl.BlockSpec((BRS, 1), lambda i: (i, 0)),
        pl.BlockSpec((1, CH), lambda i: (0, 0)),
    ],
    out_specs=pl.BlockSpec((BRS, CH), lambda i: (i, 0)),
    out_shape=jax.ShapeDtypeStruct((N_SP, CH), jnp.float32),
)


def kernel(x, edge_index, W1, b1, W2, b2):
    src = edge_index[0].astype(jnp.int32)
    dst = edge_index[1].astype(jnp.int32)
    npad = E_PAD - src.shape[0]
    pad = N_NODES + jnp.arange(npad, dtype=jnp.int32) % (N_SP - N_NODES)
    src2 = jnp.concatenate([src, pad]).reshape(NW * NBT, EPB)
    dstf = jnp.concatenate([dst, pad])
    dst2 = dstf.reshape(NW * NBT, EPB)
    xp = jnp.zeros((N_SP, CH), jnp.float32).at[:N_NODES].set(x)
    zeros_ch = jnp.zeros((N_SP, CH), jnp.float32)
    zeros_nr = jnp.zeros((NR, 128), jnp.float32)

    degs = _sc_degree(dstf, zeros_nr)
    h1p, dinv = _tc_a(degs, xp, W1)
    acc1 = _sc_scatter(h1p, src2, dst2, zeros_ch)
    h2p = _tc_b(acc1, h1p, dinv, b1.reshape(1, CH), W2)
    acc2 = _sc_scatter(h2p, src2, dst2, zeros_ch)
    outp = _tc_c(acc2, h2p, dinv, b2.reshape(1, CH))
    return outp[:N_NODES]


# speedup vs baseline: 1.0487x; 1.0487x over previous
import functools

import jax
import jax.numpy as jnp
from jax import lax
from jax.experimental import pallas as pl
from jax.experimental.pallas import tpu as pltpu
from jax.experimental.pallas import tpu_sc as plsc

N_NODES = 10000
CH = 128
N_SP = 10240
RPT = N_SP // 16
EPB = 128
NBT = 80
NW = 32
E_PAD = NW * NBT * EPB

_mesh = plsc.VectorSubcoreMesh(core_axis_name="c", subcore_axis_name="s")


EPT = E_PAD // NW
NR = 80


@functools.partial(
    pl.kernel,
    out_type=jax.ShapeDtypeStruct((2, NR, 128), jnp.float32),
    mesh=_mesh,
    scratch_types=[
        pltpu.VMEM((EPT,), jnp.int32),
        pltpu.VMEM((NR, 128), jnp.float32),
        pltpu.VMEM_SHARED((NR, 128), jnp.float32),
    ],
    compiler_params=pltpu.CompilerParams(needs_layout_passes=False),
)
def _sc_degree(dst_hbm, zeros_hbm, out_hbm, dst_v, hist_v, acc_sh):
    c = lax.axis_index("c")
    s = lax.axis_index("s")
    wid = s * 2 + c
    pltpu.sync_copy(zeros_hbm, hist_v)
    pltpu.sync_copy(dst_hbm.at[pl.ds(wid * EPT, EPT)], dst_v)

    @pl.when(s == 0)
    def _():
        pltpu.sync_copy(zeros_hbm, acc_sh)

    plsc.subcore_barrier()

    def body(i, carry):
        idx = dst_v[pl.ds(i * 16, 16)]
        cnt, last = plsc.scan_count(idx)
        row = lax.shift_right_logical(idx, 7)
        col = lax.bitwise_and(idx, 127)
        plsc.addupdate_scatter(hist_v, [row, col], cnt.astype(jnp.float32),
                               mask=last)
        return carry

    lax.fori_loop(0, EPT // 16, body, 0)
    def combine(k, carry):
        rows = jnp.arange(16, dtype=jnp.int32) + k * 16
        pltpu.sync_copy(hist_v.at[pl.ds(k * 16, 16)], acc_sh.at[rows], add=True)
        return carry

    lax.fori_loop(0, NR // 16, combine, 0)
    plsc.subcore_barrier()

    @pl.when(s < 10)
    def _():
        pltpu.sync_copy(acc_sh.at[pl.ds(s * 8, 8)],
                        out_hbm.at[c, pl.ds(s * 8, 8)])


INNER = 40
DEPTH = 2
BF = 80


@functools.partial(
    pl.kernel,
    out_type=jax.ShapeDtypeStruct((2, N_SP, CH), jnp.float32),
    mesh=_mesh,
    scratch_types=[
        pltpu.VMEM((INNER, EPB), jnp.int32),
        pltpu.VMEM((INNER, EPB), jnp.int32),
        pltpu.VMEM((EPB, CH), jnp.float32),
        pltpu.VMEM((EPB, CH), jnp.float32),
        pltpu.VMEM_SHARED((N_SP, CH), jnp.float32),
        pltpu.SemaphoreType.DMA,
        pltpu.SemaphoreType.DMA,
    ],
)
def _sc_scatter(h_hbm, src_hbm, dst_hbm, zeros_hbm, out_hbm,
                src_v, dst_v, b0, b1, acc_sh, s0, s1):
    bufs = (b0, b1)
    sems = (s0, s1)
    c = lax.axis_index("c")
    s = lax.axis_index("s")
    base = s * RPT
    with jax.named_scope("zero_acc"):
        pltpu.sync_copy(zeros_hbm.at[pl.ds(base, RPT)], acc_sh.at[pl.ds(base, RPT)])
        plsc.subcore_barrier()

    def chunk(row0):
        pltpu.sync_copy(src_hbm.at[pl.ds(row0, INNER)], src_v)
        pltpu.sync_copy(dst_hbm.at[pl.ds(row0, INNER)], dst_v)
        descs = [None] * INNER
        for b in range(DEPTH):
            descs[b] = pltpu.async_copy(h_hbm.at[src_v.at[b]], bufs[b], sems[b])
        for b in range(INNER):
            descs[b].wait()
            pltpu.sync_copy(bufs[b % DEPTH], acc_sh.at[dst_v.at[b]], add=True)
            if b + DEPTH < INNER:
                nb = b + DEPTH
                descs[nb] = pltpu.async_copy(
                    h_hbm.at[src_v.at[nb]], bufs[nb % DEPTH], sems[nb % DEPTH])

    with jax.named_scope("edge_loop"):
        @pl.when(c == 0)
        def _():
            def body0(g, carry):
                chunk(s * 2 * NBT + g * INNER)
                return carry
            lax.fori_loop(0, BF // INNER, body0, 0)

        @pl.when(c == 1)
        def _():
            def body1(g, carry):
                chunk(s * 2 * NBT + BF + g * INNER)
                return carry
            lax.fori_loop(0, (2 * NBT - BF) // INNER, body1, 0)

    with jax.named_scope("drain"):
        plsc.subcore_barrier()
    with jax.named_scope("out_copy"):
        pltpu.sync_copy(acc_sh.at[pl.ds(base, RPT)], out_hbm.at[c, pl.ds(base, RPT)])


NBLK = 8
BRS = N_SP // NBLK
NRB = N_SP // 128


def _tc_a_body(deg_ref, x_ref, w_ref, h_ref, dinv_ref):
    deg = deg_ref[0] + deg_ref[1] + 1.0
    dinv8 = lax.rsqrt(deg)
    h = jnp.dot(x_ref[...], w_ref[...], preferred_element_type=jnp.float32)
    for k in range(8):
        col = dinv8[k:k + 1, :].reshape(128, 1)
        dinv_ref[pl.ds(k * 128, 128), :] = col
        h_ref[pl.ds(k * 128, 128), :] = h[k * 128:(k + 1) * 128, :] * col


_tc_a = pl.pallas_call(
    _tc_a_body,
    grid=(10,),
    in_specs=[
        pl.BlockSpec((2, 8, 128), lambda i: (0, i, 0)),
        pl.BlockSpec((1024, CH), lambda i: (i, 0)),
        pl.BlockSpec((CH, CH), lambda i: (0, 0)),
    ],
    out_specs=(
        pl.BlockSpec((1024, CH), lambda i: (i, 0)),
        pl.BlockSpec((1024, 1), lambda i: (i, 0)),
    ),
    out_shape=(
        jax.ShapeDtypeStruct((N_SP, CH), jnp.float32),
        jax.ShapeDtypeStruct((N_SP, 1), jnp.float32),
    ),
)


def _tc_b_body(acc_ref, h_ref, dinv_ref, b_ref, w_ref, out_ref):
    dinv = dinv_ref[...]
    z = (acc_ref[0] + acc_ref[1] + h_ref[...]) * dinv + b_ref[...]
    z = jnp.maximum(z, 0.0)
    out_ref[...] = jnp.dot(z, w_ref[...], preferred_element_type=jnp.float32) * dinv


_tc_b = pl.pallas_call(
    _tc_b_body,
    grid=(NBLK,),
    in_specs=[
        pl.BlockSpec((2, BRS, CH), lambda i: (0, i, 0)),
        pl.BlockSpec((BRS, CH), lambda i: (i, 0)),
        pl.BlockSpec((BRS, 1), lambda i: (i, 0)),
        pl.BlockSpec((1, CH), lambda i: (0, 0)),
        pl.BlockSpec((CH, CH), lambda i: (0, 0)),
    ],
    out_specs=pl.BlockSpec((BRS, CH), lambda i: (i, 0)),
    out_shape=jax.ShapeDtypeStruct((N_SP, CH), jnp.float32),
)


def _tc_c_body(acc_ref, h_ref, dinv_ref, b_ref, out_ref):
    out_ref[...] = (acc_ref[0] + acc_ref[1] + h_ref[...]) * dinv_ref[...] + b_ref[...]


_tc_c = pl.pallas_call(
    _tc_c_body,
    grid=(NBLK,),
    in_specs=[
        pl.BlockSpec((2, BRS, CH), lambda i: (0, i, 0)),
        pl.BlockSpec((BRS, CH), lambda i: (i, 0)),
        pl.BlockSpec((BRS, 1), lambda i: (i, 0)),
        pl.BlockSpec((1, CH), lambda i: (0, 0)),
    ],
    out_specs=pl.BlockSpec((BRS, CH), lambda i: (i, 0)),
    out_shape=jax.ShapeDtypeStruct((N_SP, CH), jnp.float32),
)


def kernel(x, edge_index, W1, b1, W2, b2):
    src = edge_index[0].astype(jnp.int32)
    dst = edge_index[1].astype(jnp.int32)
    npad = E_PAD - src.shape[0]
    pad = N_NODES + jnp.arange(npad, dtype=jnp.int32) % (N_SP - N_NODES)
    src2 = jnp.concatenate([src, pad]).reshape(NW * NBT, EPB)
    dstf = jnp.concatenate([dst, pad])
    dst2 = dstf.reshape(NW * NBT, EPB)
    xp = jnp.zeros((N_SP, CH), jnp.float32).at[:N_NODES].set(x)
    zeros_ch = jnp.zeros((N_SP, CH), jnp.float32)
    zeros_nr = jnp.zeros((NR, 128), jnp.float32)

    degs = _sc_degree(dstf, zeros_nr)
    h1p, dinv = _tc_a(degs, xp, W1)
    acc1 = _sc_scatter(h1p, src2, dst2, zeros_ch)
    h2p = _tc_b(acc1, h1p, dinv, b1.reshape(1, CH), W2)
    acc2 = _sc_scatter(h2p, src2, dst2, zeros_ch)
    outp = _tc_c(acc2, h2p, dinv, b2.reshape(1, CH))
    return outp[:N_NODES]


# speedup vs baseline: 1.0523x; 1.0035x over previous
import functools

import jax
import jax.numpy as jnp
from jax import lax
from jax.experimental import pallas as pl
from jax.experimental.pallas import tpu as pltpu
from jax.experimental.pallas import tpu_sc as plsc

N_NODES = 10000
CH = 128
N_SP = 10240
RPT = N_SP // 16
EPB = 128
NBT = 80
NW = 32
E_PAD = NW * NBT * EPB

_mesh = plsc.VectorSubcoreMesh(core_axis_name="c", subcore_axis_name="s")


EPT = E_PAD // NW
NR = 80


@functools.partial(
    pl.kernel,
    out_type=jax.ShapeDtypeStruct((2, NR, 128), jnp.float32),
    mesh=_mesh,
    scratch_types=[
        pltpu.VMEM((EPT,), jnp.int32),
        pltpu.VMEM((NR, 128), jnp.float32),
        pltpu.VMEM_SHARED((NR, 128), jnp.float32),
    ],
    compiler_params=pltpu.CompilerParams(needs_layout_passes=False),
)
def _sc_degree(dst_hbm, zeros_hbm, out_hbm, dst_v, hist_v, acc_sh):
    c = lax.axis_index("c")
    s = lax.axis_index("s")
    wid = s * 2 + c
    pltpu.sync_copy(zeros_hbm, hist_v)
    pltpu.sync_copy(dst_hbm.at[pl.ds(wid * EPT, EPT)], dst_v)

    @pl.when(s == 0)
    def _():
        pltpu.sync_copy(zeros_hbm, acc_sh)

    plsc.subcore_barrier()

    def body(i, carry):
        idx = dst_v[pl.ds(i * 16, 16)]
        cnt, last = plsc.scan_count(idx)
        row = lax.shift_right_logical(idx, 7)
        col = lax.bitwise_and(idx, 127)
        plsc.addupdate_scatter(hist_v, [row, col], cnt.astype(jnp.float32),
                               mask=last)
        return carry

    lax.fori_loop(0, EPT // 16, body, 0)
    def combine(k, carry):
        rows = jnp.arange(16, dtype=jnp.int32) + k * 16
        pltpu.sync_copy(hist_v.at[pl.ds(k * 16, 16)], acc_sh.at[rows], add=True)
        return carry

    lax.fori_loop(0, NR // 16, combine, 0)
    plsc.subcore_barrier()

    @pl.when(s < 10)
    def _():
        pltpu.sync_copy(acc_sh.at[pl.ds(s * 8, 8)],
                        out_hbm.at[c, pl.ds(s * 8, 8)])


INNER = 40
DEPTH = 2
BF = 80


@functools.partial(
    pl.kernel,
    out_type=jax.ShapeDtypeStruct((2, N_SP, CH), jnp.float32),
    mesh=_mesh,
    scratch_types=[
        pltpu.VMEM((INNER, EPB), jnp.int32),
        pltpu.VMEM((INNER, EPB), jnp.int32),
        pltpu.VMEM((EPB, CH), jnp.float32),
        pltpu.VMEM((EPB, CH), jnp.float32),
        pltpu.VMEM_SHARED((N_SP, CH), jnp.float32),
        pltpu.SemaphoreType.DMA,
        pltpu.SemaphoreType.DMA,
    ],
)
def _sc_scatter(h_hbm, src_hbm, dst_hbm, zeros_hbm, out_hbm,
                src_v, dst_v, b0, b1, acc_sh, s0, s1):
    bufs = (b0, b1)
    sems = (s0, s1)
    c = lax.axis_index("c")
    s = lax.axis_index("s")
    base = s * RPT
    with jax.named_scope("zero_acc"):
        pltpu.sync_copy(zeros_hbm.at[pl.ds(base, RPT)], acc_sh.at[pl.ds(base, RPT)])
        plsc.subcore_barrier()

    def chunk(row0):
        pltpu.sync_copy(src_hbm.at[pl.ds(row0, INNER)], src_v)
        pltpu.sync_copy(dst_hbm.at[pl.ds(row0, INNER)], dst_v)
        descs = [None] * INNER
        for b in range(DEPTH):
            descs[b] = pltpu.async_copy(h_hbm.at[src_v.at[b]], bufs[b], sems[b])
        for b in range(INNER):
            descs[b].wait()
            pltpu.sync_copy(bufs[b % DEPTH], acc_sh.at[dst_v.at[b]], add=True)
            if b + DEPTH < INNER:
                nb = b + DEPTH
                descs[nb] = pltpu.async_copy(
                    h_hbm.at[src_v.at[nb]], bufs[nb % DEPTH], sems[nb % DEPTH])

    with jax.named_scope("edge_loop"):
        @pl.when(c == 0)
        def _():
            def body0(g, carry):
                chunk(s * 2 * NBT + g * INNER)
                return carry
            lax.fori_loop(0, BF // INNER, body0, 0)

        @pl.when(c == 1)
        def _():
            def body1(g, carry):
                chunk(s * 2 * NBT + BF + g * INNER)
                return carry
            lax.fori_loop(0, (2 * NBT - BF) // INNER, body1, 0)

    with jax.named_scope("drain"):
        plsc.subcore_barrier()
    with jax.named_scope("out_copy"):
        pltpu.sync_copy(acc_sh.at[pl.ds(base, RPT)], out_hbm.at[c, pl.ds(base, RPT)])


NBLK = 8
BRS = N_SP // NBLK


def _tc_a_body(deg_ref, x_ref, w_ref, h_ref, dinv_ref):
    deg = deg_ref[0] + deg_ref[1] + 1.0
    dinv8 = lax.rsqrt(deg)
    h = jnp.dot(x_ref[...], w_ref[...], preferred_element_type=jnp.float32)
    for k in range(8):
        col = dinv8[k:k + 1, :].reshape(128, 1)
        dinv_ref[pl.ds(k * 128, 128), :] = col
        h_ref[pl.ds(k * 128, 128), :] = h[k * 128:(k + 1) * 128, :] * col


_tc_a = pl.pallas_call(
    _tc_a_body,
    grid=(10,),
    in_specs=[
        pl.BlockSpec((2, 8, 128), lambda i: (0, i, 0)),
        pl.BlockSpec((1024, CH), lambda i: (i, 0)),
        pl.BlockSpec((CH, CH), lambda i: (0, 0)),
    ],
    out_specs=(
        pl.BlockSpec((1024, CH), lambda i: (i, 0)),
        pl.BlockSpec((1024, 1), lambda i: (i, 0)),
    ),
    out_shape=(
        jax.ShapeDtypeStruct((N_SP, CH), jnp.float32),
        jax.ShapeDtypeStruct((N_SP, 1), jnp.float32),
    ),
)


def _tc_b_body(acc_ref, h_ref, dinv_ref, b_ref, w_ref, out_ref):
    dinv = dinv_ref[...]
    z = (acc_ref[0] + acc_ref[1] + h_ref[...]) * dinv + b_ref[...]
    z = jnp.maximum(z, 0.0)
    out_ref[...] = jnp.dot(z, w_ref[...], preferred_element_type=jnp.float32) * dinv


_tc_b = pl.pallas_call(
    _tc_b_body,
    grid=(NBLK,),
    in_specs=[
        pl.BlockSpec((2, BRS, CH), lambda i: (0, i, 0)),
        pl.BlockSpec((BRS, CH), lambda i: (i, 0)),
        pl.BlockSpec((BRS, 1), lambda i: (i, 0)),
        pl.BlockSpec((1, CH), lambda i: (0, 0)),
        pl.BlockSpec((CH, CH), lambda i: (0, 0)),
    ],
    out_specs=pl.BlockSpec((BRS, CH), lambda i: (i, 0)),
    out_shape=jax.ShapeDtypeStruct((N_SP, CH), jnp.float32),
)


def _tc_c_body(acc_ref, h_ref, dinv_ref, b_ref, out_ref):
    out_ref[...] = (acc_ref[0] + acc_ref[1] + h_ref[...]) * dinv_ref[...] + b_ref[...]


_tc_c = pl.pallas_call(
    _tc_c_body,
    grid=(NBLK,),
    in_specs=[
        pl.BlockSpec((2, BRS, CH), lambda i: (0, i, 0)),
        pl.BlockSpec((BRS, CH), lambda i: (i, 0)),
        pl.BlockSpec((BRS, 1), lambda i: (i, 0)),
        pl.BlockSpec((1, CH), lambda i: (0, 0)),
    ],
    out_specs=pl.BlockSpec((BRS, CH), lambda i: (i, 0)),
    out_shape=jax.ShapeDtypeStruct((N_SP, CH), jnp.float32),
)


def kernel(x, edge_index, W1, b1, W2, b2):
    src = edge_index[0].astype(jnp.int32)
    dst = edge_index[1].astype(jnp.int32)
    npad = E_PAD - src.shape[0]
    pad = N_NODES + jnp.arange(npad, dtype=jnp.int32) % (N_SP - N_NODES)
    src2 = jnp.concatenate([src, pad]).reshape(NW * NBT, EPB)
    dstf = jnp.concatenate([dst, pad])
    dst2 = dstf.reshape(NW * NBT, EPB)
    xp = jnp.zeros((N_SP, CH), jnp.float32).at[:N_NODES].set(x)
    zeros_ch = jnp.zeros((N_SP, CH), jnp.float32)
    zeros_nr = jnp.zeros((NR, 128), jnp.float32)

    degs = _sc_degree(dstf, zeros_nr)
    h1p, dinv = _tc_a(degs, xp, W1)
    acc1 = _sc_scatter(h1p, src2, dst2, zeros_ch)
    h2p = _tc_b(acc1, h1p, dinv, b1.reshape(1, CH), W2)
    acc2 = _sc_scatter(h2p, src2, dst2, zeros_ch)
    outp = _tc_c(acc2, h2p, dinv, b2.reshape(1, CH))
    return outp[:N_NODES]


# speedup vs baseline: 1.0720x; 1.0187x over previous
import functools

import jax
import jax.numpy as jnp
from jax import lax
from jax.experimental import pallas as pl
from jax.experimental.pallas import tpu as pltpu
from jax.experimental.pallas import tpu_sc as plsc

N_NODES = 10000
CH = 128
N_SP = 10240
RPT = N_SP // 16
EPB = 128
NBT = 80
NW = 32
E_PAD = NW * NBT * EPB

_mesh = plsc.VectorSubcoreMesh(core_axis_name="c", subcore_axis_name="s")


EPT = E_PAD // NW
NR = 80


@functools.partial(
    pl.kernel,
    out_type=jax.ShapeDtypeStruct((2, NR, 128), jnp.float32),
    mesh=_mesh,
    scratch_types=[
        pltpu.VMEM((EPT,), jnp.int32),
        pltpu.VMEM((NR, 128), jnp.float32),
        pltpu.VMEM_SHARED((NR, 128), jnp.float32),
    ],
    compiler_params=pltpu.CompilerParams(needs_layout_passes=False),
)
def _sc_degree(dst_hbm, zeros_hbm, out_hbm, dst_v, hist_v, acc_sh):
    c = lax.axis_index("c")
    s = lax.axis_index("s")
    wid = s * 2 + c
    pltpu.sync_copy(zeros_hbm, hist_v)
    pltpu.sync_copy(dst_hbm.at[pl.ds(wid * EPT, EPT)], dst_v)

    @pl.when(s == 0)
    def _():
        pltpu.sync_copy(zeros_hbm, acc_sh)

    plsc.subcore_barrier()

    def body(i, carry):
        idx = dst_v[pl.ds(i * 16, 16)]
        cnt, last = plsc.scan_count(idx)
        row = lax.shift_right_logical(idx, 7)
        col = lax.bitwise_and(idx, 127)
        plsc.addupdate_scatter(hist_v, [row, col], cnt.astype(jnp.float32),
                               mask=last)
        return carry

    lax.fori_loop(0, EPT // 16, body, 0)
    def combine(k, carry):
        rows = jnp.arange(16, dtype=jnp.int32) + k * 16
        pltpu.sync_copy(hist_v.at[pl.ds(k * 16, 16)], acc_sh.at[rows], add=True)
        return carry

    lax.fori_loop(0, NR // 16, combine, 0)
    plsc.subcore_barrier()

    @pl.when(s < 10)
    def _():
        pltpu.sync_copy(acc_sh.at[pl.ds(s * 8, 8)],
                        out_hbm.at[c, pl.ds(s * 8, 8)])


INNER = 40
DEPTH = 2
BF = 80


@functools.partial(
    pl.kernel,
    out_type=jax.ShapeDtypeStruct((2, N_SP, CH), jnp.float32),
    mesh=_mesh,
    scratch_types=[
        pltpu.VMEM((INNER, EPB), jnp.int32),
        pltpu.VMEM((INNER, EPB), jnp.int32),
        pltpu.VMEM((EPB, CH), jnp.float32),
        pltpu.VMEM((EPB, CH), jnp.float32),
        pltpu.VMEM_SHARED((N_SP, CH), jnp.float32),
        pltpu.SemaphoreType.DMA,
        pltpu.SemaphoreType.DMA,
        pltpu.SemaphoreType.DMA,
    ],
)
def _sc_scatter(h_hbm, src_hbm, dst_hbm, zeros_hbm, out_hbm,
                src_v, dst_v, b0, b1, acc_sh, s0, s1, zsem):
    bufs = (b0, b1)
    sems = (s0, s1)
    c = lax.axis_index("c")
    s = lax.axis_index("s")
    base = s * RPT
    row_base = s * 2 * NBT + (2 * NBT - BF) * c
    zdesc = pltpu.async_copy(zeros_hbm.at[pl.ds(base, RPT)],
                             acc_sh.at[pl.ds(base, RPT)], zsem)

    def load_idx(row0):
        pltpu.sync_copy(src_hbm.at[pl.ds(row0, INNER)], src_v)
        pltpu.sync_copy(dst_hbm.at[pl.ds(row0, INNER)], dst_v)

    def run_chunk(first):
        descs = [None] * INNER
        for b in range(DEPTH):
            descs[b] = pltpu.async_copy(h_hbm.at[src_v.at[b]], bufs[b], sems[b])
        if first:
            zdesc.wait()
            plsc.subcore_barrier()
        for b in range(INNER):
            descs[b].wait()
            pltpu.sync_copy(bufs[b % DEPTH], acc_sh.at[dst_v.at[b]], add=True)
            if b + DEPTH < INNER:
                nb = b + DEPTH
                descs[nb] = pltpu.async_copy(
                    h_hbm.at[src_v.at[nb]], bufs[nb % DEPTH], sems[nb % DEPTH])

    load_idx(row_base)
    run_chunk(first=True)
    load_idx(row_base + INNER)
    run_chunk(first=False)

    plsc.subcore_barrier()
    pltpu.sync_copy(acc_sh.at[pl.ds(base, RPT)], out_hbm.at[c, pl.ds(base, RPT)])


NBLK = 8
BRS = N_SP // NBLK


def _tc_a_body(deg_ref, x_ref, w_ref, h_ref, dinv_ref):
    deg = deg_ref[0] + deg_ref[1] + 1.0
    dinv8 = lax.rsqrt(deg)
    h = jnp.dot(x_ref[...], w_ref[...], preferred_element_type=jnp.float32)
    for k in range(8):
        col = dinv8[k:k + 1, :].reshape(128, 1)
        dinv_ref[pl.ds(k * 128, 128), :] = col
        h_ref[pl.ds(k * 128, 128), :] = h[k * 128:(k + 1) * 128, :] * col


_tc_a = pl.pallas_call(
    _tc_a_body,
    grid=(10,),
    in_specs=[
        pl.BlockSpec((2, 8, 128), lambda i: (0, i, 0)),
        pl.BlockSpec((1024, CH), lambda i: (i, 0)),
        pl.BlockSpec((CH, CH), lambda i: (0, 0)),
    ],
    out_specs=(
        pl.BlockSpec((1024, CH), lambda i: (i, 0)),
        pl.BlockSpec((1024, 1), lambda i: (i, 0)),
    ),
    out_shape=(
        jax.ShapeDtypeStruct((N_SP, CH), jnp.float32),
        jax.ShapeDtypeStruct((N_SP, 1), jnp.float32),
    ),
)


def _tc_b_body(acc_ref, h_ref, dinv_ref, b_ref, w_ref, out_ref):
    dinv = dinv_ref[...]
    z = (acc_ref[0] + acc_ref[1] + h_ref[...]) * dinv + b_ref[...]
    z = jnp.maximum(z, 0.0)
    out_ref[...] = jnp.dot(z, w_ref[...], preferred_element_type=jnp.float32) * dinv


_tc_b = pl.pallas_call(
    _tc_b_body,
    grid=(NBLK,),
    in_specs=[
        pl.BlockSpec((2, BRS, CH), lambda i: (0, i, 0)),
        pl.BlockSpec((BRS, CH), lambda i: (i, 0)),
        pl.BlockSpec((BRS, 1), lambda i: (i, 0)),
        pl.BlockSpec((1, CH), lambda i: (0, 0)),
        pl.BlockSpec((CH, CH), lambda i: (0, 0)),
    ],
    out_specs=pl.BlockSpec((BRS, CH), lambda i: (i, 0)),
    out_shape=jax.ShapeDtypeStruct((N_SP, CH), jnp.float32),
)


def _tc_c_body(acc_ref, h_ref, dinv_ref, b_ref, out_ref):
    out_ref[...] = (acc_ref[0] + acc_ref[1] + h_ref[...]) * dinv_ref[...] + b_ref[...]


_tc_c = pl.pallas_call(
    _tc_c_body,
    grid=(NBLK,),
    in_specs=[
        pl.BlockSpec((2, BRS, CH), lambda i: (0, i, 0)),
        pl.BlockSpec((BRS, CH), lambda i: (i, 0)),
        pl.BlockSpec((BRS, 1), lambda i: (i, 0)),
        pl.BlockSpec((1, CH), lambda i: (0, 0)),
    ],
    out_specs=pl.BlockSpec((BRS, CH), lambda i: (i, 0)),
    out_shape=jax.ShapeDtypeStruct((N_SP, CH), jnp.float32),
)


def kernel(x, edge_index, W1, b1, W2, b2):
    src = edge_index[0].astype(jnp.int32)
    dst = edge_index[1].astype(jnp.int32)
    npad = E_PAD - src.shape[0]
    pad = N_NODES + jnp.arange(npad, dtype=jnp.int32) % (N_SP - N_NODES)
    src2 = jnp.concatenate([src, pad]).reshape(NW * NBT, EPB)
    dstf = jnp.concatenate([dst, pad])
    dst2 = dstf.reshape(NW * NBT, EPB)
    xp = jnp.zeros((N_SP, CH), jnp.float32).at[:N_NODES].set(x)
    zeros_ch = jnp.zeros((N_SP, CH), jnp.float32)
    zeros_nr = jnp.zeros((NR, 128), jnp.float32)

    degs = _sc_degree(dstf, zeros_nr)
    h1p, dinv = _tc_a(degs, xp, W1)
    acc1 = _sc_scatter(h1p, src2, dst2, zeros_ch)
    h2p = _tc_b(acc1, h1p, dinv, b1.reshape(1, CH), W2)
    acc2 = _sc_scatter(h2p, src2, dst2, zeros_ch)
    outp = _tc_c(acc2, h2p, dinv, b2.reshape(1, CH))
    return outp[:N_NODES]


# speedup vs baseline: 1.0862x; 1.0132x over previous
import functools

import jax
import jax.numpy as jnp
from jax import lax
from jax.experimental import pallas as pl
from jax.experimental.pallas import tpu as pltpu
from jax.experimental.pallas import tpu_sc as plsc

N_NODES = 10000
CH = 128
N_SP = 10240
RPT = N_SP // 16
EPB = 128
NBT = 80
NW = 32
E_PAD = NW * NBT * EPB

_mesh = plsc.VectorSubcoreMesh(core_axis_name="c", subcore_axis_name="s")


EPT = E_PAD // NW
NR = 80


@functools.partial(
    pl.kernel,
    out_type=jax.ShapeDtypeStruct((2, NR, 128), jnp.float32),
    mesh=_mesh,
    scratch_types=[
        pltpu.VMEM((EPT,), jnp.int32),
        pltpu.VMEM((NR, 128), jnp.float32),
        pltpu.VMEM_SHARED((NR, 128), jnp.float32),
    ],
    compiler_params=pltpu.CompilerParams(needs_layout_passes=False),
)
def _sc_degree(dst_hbm, zeros_hbm, out_hbm, dst_v, hist_v, acc_sh):
    c = lax.axis_index("c")
    s = lax.axis_index("s")
    wid = s * 2 + c
    pltpu.sync_copy(zeros_hbm, hist_v)
    pltpu.sync_copy(dst_hbm.at[pl.ds(wid * EPT, EPT)], dst_v)

    @pl.when(s == 0)
    def _():
        pltpu.sync_copy(zeros_hbm, acc_sh)

    plsc.subcore_barrier()

    def body(i, carry):
        idx = dst_v[pl.ds(i * 16, 16)]
        cnt, last = plsc.scan_count(idx)
        row = lax.shift_right_logical(idx, 7)
        col = lax.bitwise_and(idx, 127)
        plsc.addupdate_scatter(hist_v, [row, col], cnt.astype(jnp.float32),
                               mask=last)
        return carry

    lax.fori_loop(0, EPT // 16, body, 0)
    def combine(k, carry):
        rows = jnp.arange(16, dtype=jnp.int32) + k * 16
        pltpu.sync_copy(hist_v.at[pl.ds(k * 16, 16)], acc_sh.at[rows], add=True)
        return carry

    lax.fori_loop(0, NR // 16, combine, 0)
    plsc.subcore_barrier()

    @pl.when(s < 10)
    def _():
        pltpu.sync_copy(acc_sh.at[pl.ds(s * 8, 8)],
                        out_hbm.at[c, pl.ds(s * 8, 8)])


INNER = 40
DEPTH = 2
BF = 80


@functools.partial(
    pl.kernel,
    out_type=jax.ShapeDtypeStruct((2, N_SP, CH), jnp.float32),
    mesh=_mesh,
    scratch_types=[
        pltpu.VMEM((INNER, EPB), jnp.int32),
        pltpu.VMEM((INNER, EPB), jnp.int32),
        pltpu.VMEM((EPB, CH), jnp.float32),
        pltpu.VMEM((EPB, CH), jnp.float32),
        pltpu.VMEM_SHARED((N_SP, CH), jnp.float32),
        pltpu.SemaphoreType.DMA,
        pltpu.SemaphoreType.DMA,
        pltpu.SemaphoreType.DMA,
    ],
)
def _sc_scatter(h_hbm, src_hbm, dst_hbm, zeros_hbm, out_hbm,
                src_v, dst_v, b0, b1, acc_sh, s0, s1, zsem):
    bufs = (b0, b1)
    sems = (s0, s1)
    c = lax.axis_index("c")
    s = lax.axis_index("s")
    base = s * RPT
    row_base = s * 2 * NBT + (2 * NBT - BF) * c
    zdesc = pltpu.async_copy(zeros_hbm.at[pl.ds(base, RPT)],
                             acc_sh.at[pl.ds(base, RPT)], zsem)

    def load_idx(row0):
        pltpu.sync_copy(src_hbm.at[pl.ds(row0, INNER)], src_v)
        pltpu.sync_copy(dst_hbm.at[pl.ds(row0, INNER)], dst_v)

    def run_chunk(first):
        descs = [None] * INNER
        for b in range(DEPTH):
            descs[b] = pltpu.async_copy(h_hbm.at[src_v.at[b]], bufs[b], sems[b])
        if first:
            zdesc.wait()
            plsc.subcore_barrier()
        for b in range(INNER):
            descs[b].wait()
            pltpu.sync_copy(bufs[b % DEPTH], acc_sh.at[dst_v.at[b]], add=True)
            if b + DEPTH < INNER:
                nb = b + DEPTH
                descs[nb] = pltpu.async_copy(
                    h_hbm.at[src_v.at[nb]], bufs[nb % DEPTH], sems[nb % DEPTH])

    load_idx(row_base)
    run_chunk(first=True)
    load_idx(row_base + INNER)
    run_chunk(first=False)

    plsc.subcore_barrier()
    pltpu.sync_copy(acc_sh.at[pl.ds(base, RPT)], out_hbm.at[c, pl.ds(base, RPT)])


NBLK = 8
BRS = N_SP // NBLK


def _tc_a_body(deg_ref, x_ref, w_ref, h_ref, dinv_ref):
    deg = deg_ref[0] + deg_ref[1] + 1.0
    dinv8 = lax.rsqrt(deg)
    h = jnp.dot(x_ref[...], w_ref[...], preferred_element_type=jnp.float32)
    for k in range(8):
        col = dinv8[k:k + 1, :].reshape(128, 1)
        dinv_ref[pl.ds(k * 128, 128), :] = col
        h_ref[pl.ds(k * 128, 128), :] = h[k * 128:(k + 1) * 128, :] * col


_tc_a = pl.pallas_call(
    _tc_a_body,
    grid=(10,),
    in_specs=[
        pl.BlockSpec((2, 8, 128), lambda i: (0, i, 0)),
        pl.BlockSpec((1024, CH), lambda i: (i, 0)),
        pl.BlockSpec((CH, CH), lambda i: (0, 0)),
    ],
    out_specs=(
        pl.BlockSpec((1024, CH), lambda i: (i, 0)),
        pl.BlockSpec((1024, 1), lambda i: (i, 0)),
    ),
    out_shape=(
        jax.ShapeDtypeStruct((N_SP, CH), jnp.float32),
        jax.ShapeDtypeStruct((N_SP, 1), jnp.float32),
    ),
)


def _tc_b_body(acc_ref, h_ref, dinv_ref, b_ref, w_ref, out_ref):
    dinv = dinv_ref[...]
    z = (acc_ref[0] + acc_ref[1] + h_ref[...]) * dinv + b_ref[...]
    z = jnp.maximum(z, 0.0)
    out_ref[...] = jnp.dot(z, w_ref[...], preferred_element_type=jnp.float32) * dinv


_tc_b = pl.pallas_call(
    _tc_b_body,
    grid=(NBLK,),
    in_specs=[
        pl.BlockSpec((2, BRS, CH), lambda i: (0, i, 0)),
        pl.BlockSpec((BRS, CH), lambda i: (i, 0)),
        pl.BlockSpec((BRS, 1), lambda i: (i, 0)),
        pl.BlockSpec((1, CH), lambda i: (0, 0)),
        pl.BlockSpec((CH, CH), lambda i: (0, 0)),
    ],
    out_specs=pl.BlockSpec((BRS, CH), lambda i: (i, 0)),
    out_shape=jax.ShapeDtypeStruct((N_SP, CH), jnp.float32),
)


def _tc_c_body(acc_ref, h_ref, dinv_ref, b_ref, out_ref):
    n = pl.ds(0, N_NODES)
    acc = acc_ref[0, n, :] + acc_ref[1, n, :] + h_ref[n, :]
    out_ref[...] = acc * dinv_ref[n, :] + b_ref[...]


_tc_c = pl.pallas_call(
    _tc_c_body,
    out_shape=jax.ShapeDtypeStruct((N_NODES, CH), jnp.float32),
)


def kernel(x, edge_index, W1, b1, W2, b2):
    src = edge_index[0].astype(jnp.int32)
    dst = edge_index[1].astype(jnp.int32)
    npad = E_PAD - src.shape[0]
    pad = N_NODES + jnp.arange(npad, dtype=jnp.int32) % (N_SP - N_NODES)
    src2 = jnp.concatenate([src, pad]).reshape(NW * NBT, EPB)
    dstf = jnp.concatenate([dst, pad])
    dst2 = dstf.reshape(NW * NBT, EPB)
    xp = jnp.zeros((N_SP, CH), jnp.float32).at[:N_NODES].set(x)
    zeros_ch = jnp.zeros((N_SP, CH), jnp.float32)
    zeros_nr = jnp.zeros((NR, 128), jnp.float32)

    degs = _sc_degree(dstf, zeros_nr)
    h1p, dinv = _tc_a(degs, xp, W1)
    acc1 = _sc_scatter(h1p, src2, dst2, zeros_ch)
    h2p = _tc_b(acc1, h1p, dinv, b1.reshape(1, CH), W2)
    acc2 = _sc_scatter(h2p, src2, dst2, zeros_ch)
    return _tc_c(acc2, h2p, dinv, b2.reshape(1, CH))
